# baseline (speedup 1.0000x reference)
import jax
import jax.numpy as jnp
from jax.experimental import pallas as pl
from jax.experimental.pallas import tpu as pltpu

_ROW_TILE = 2512


def _body(x_ref, wz_ref, bz_ref, wh_ref, bh_ref, wl_ref, bl_ref,
          out_ref, h_ref, wcat_ref, bcat_ref):
    in_dim = x_ref.shape[1]
    hid = h_ref.shape[1]

    @pl.when(pl.program_id(0) == 0)
    def _fold_weights():
        wcat_ref[:, :hid] = (wz_ref[0, 0, :in_dim, :]
                             + wz_ref[1, 0, :in_dim, :])
        wcat_ref[:, hid:] = 2.0 * (wh_ref[0, 0, :in_dim, :]
                                   + wh_ref[1, 0, :in_dim, :])
        bcat_ref[:, :hid] = bz_ref[...]
        bcat_ref[:, hid:] = 2.0 * bh_ref[...]

    xb = x_ref[...]
    s = jax.nn.sigmoid(
        jnp.dot(xb, wcat_ref[...], preferred_element_type=jnp.float32)
        + bcat_ref[...])
    z = s[:, :hid]
    h_tilde = 2.0 * s[:, hid:] - 1.0
    h = (1.0 - z) * h_tilde
    h_ref[...] = h
    out_ref[...] = (
        jnp.dot(jnp.maximum(h, 0.0), wl_ref[...],
                preferred_element_type=jnp.float32)
        + bl_ref[...])


def kernel(x, edge_index, edge_weight, W_z, b_z, W_r, b_r, W_h, b_h,
           W_lin, b_lin):
    n, in_dim = x.shape
    cat_dim = W_z.shape[2]
    hid = W_z.shape[-1]
    out_dim = W_lin.shape[-1]

    out, h = pl.pallas_call(
        _body,
        grid=(pl.cdiv(n, _ROW_TILE),),
        in_specs=[
            pl.BlockSpec((_ROW_TILE, in_dim), lambda i: (i, 0)),
            pl.BlockSpec((2, 1, cat_dim, hid), lambda i: (0, 0, 0, 0)),
            pl.BlockSpec((1, hid), lambda i: (0, 0)),
            pl.BlockSpec((2, 1, cat_dim, hid), lambda i: (0, 0, 0, 0)),
            pl.BlockSpec((1, hid), lambda i: (0, 0)),
            pl.BlockSpec((hid, out_dim), lambda i: (0, 0)),
            pl.BlockSpec((1, out_dim), lambda i: (0, 0)),
        ],
        out_specs=[
            pl.BlockSpec((_ROW_TILE, out_dim), lambda i: (i, 0)),
            pl.BlockSpec((_ROW_TILE, hid), lambda i: (i, 0)),
        ],
        out_shape=[
            jax.ShapeDtypeStruct((n, out_dim), x.dtype),
            jax.ShapeDtypeStruct((n, hid), x.dtype),
        ],
        scratch_shapes=[
            pltpu.VMEM((in_dim, 2 * hid), jnp.float32),
            pltpu.VMEM((1, 2 * hid), jnp.float32),
        ],
    )(x, W_z, b_z.reshape(1, hid), W_h, b_h.reshape(1, hid),
      W_lin, b_lin.reshape(1, out_dim))
    return (out, h)


# speedup vs baseline: 1.0440x; 1.0440x over previous
import jax
import jax.numpy as jnp
from jax.experimental import pallas as pl
from jax.experimental.pallas import tpu as pltpu

_ROW_TILE = 5120


def _body(x_ref, wz_ref, bz_ref, wh_ref, bh_ref, wl_ref, bl_ref,
          out_ref, h_ref, wcat_ref, bcat_ref):
    in_dim = x_ref.shape[1]
    hid = h_ref.shape[1]

    @pl.when(pl.program_id(0) == 0)
    def _fold_weights():
        wcat_ref[:, :hid] = (wz_ref[0, 0, :in_dim, :]
                             + wz_ref[1, 0, :in_dim, :])
        wcat_ref[:, hid:] = 2.0 * (wh_ref[0, 0, :in_dim, :]
                                   + wh_ref[1, 0, :in_dim, :])
        bcat_ref[:, :hid] = bz_ref[...]
        bcat_ref[:, hid:] = 2.0 * bh_ref[...]

    xb = x_ref[...]
    s = jax.nn.sigmoid(
        jnp.dot(xb, wcat_ref[...], preferred_element_type=jnp.float32)
        + bcat_ref[...])
    z = s[:, :hid]
    h_tilde = 2.0 * s[:, hid:] - 1.0
    h = (1.0 - z) * h_tilde
    h_ref[...] = h
    out_ref[...] = (
        jnp.dot(jnp.maximum(h, 0.0), wl_ref[...],
                preferred_element_type=jnp.float32)
        + bl_ref[...])


def kernel(x, edge_index, edge_weight, W_z, b_z, W_r, b_r, W_h, b_h,
           W_lin, b_lin):
    n, in_dim = x.shape
    cat_dim = W_z.shape[2]
    hid = W_z.shape[-1]
    out_dim = W_lin.shape[-1]

    out, h = pl.pallas_call(
        _body,
        grid=(pl.cdiv(n, _ROW_TILE),),
        in_specs=[
            pl.BlockSpec((_ROW_TILE, in_dim), lambda i: (i, 0)),
            pl.BlockSpec((2, 1, cat_dim, hid), lambda i: (0, 0, 0, 0)),
            pl.BlockSpec((1, hid), lambda i: (0, 0)),
            pl.BlockSpec((2, 1, cat_dim, hid), lambda i: (0, 0, 0, 0)),
            pl.BlockSpec((1, hid), lambda i: (0, 0)),
            pl.BlockSpec((hid, out_dim), lambda i: (0, 0)),
            pl.BlockSpec((1, out_dim), lambda i: (0, 0)),
        ],
        out_specs=[
            pl.BlockSpec((_ROW_TILE, out_dim), lambda i: (i, 0)),
            pl.BlockSpec((_ROW_TILE, hid), lambda i: (i, 0)),
        ],
        out_shape=[
            jax.ShapeDtypeStruct((n, out_dim), x.dtype),
            jax.ShapeDtypeStruct((n, hid), x.dtype),
        ],
        scratch_shapes=[
            pltpu.VMEM((in_dim, 2 * hid), jnp.float32),
            pltpu.VMEM((1, 2 * hid), jnp.float32),
        ],
    )(x, W_z, b_z.reshape(1, hid), W_h, b_h.reshape(1, hid),
      W_lin, b_lin.reshape(1, out_dim))
    return (out, h)


# speedup vs baseline: 1.0500x; 1.0057x over previous
import jax
import jax.numpy as jnp
from jax.experimental import pallas as pl
from jax.experimental.pallas import tpu as pltpu

_ROW_TILE = 5008


def _body(x_ref, wz_ref, bz_ref, wh_ref, bh_ref, wl_ref, bl_ref,
          out_ref, h_ref, wcat_ref, bcat_ref):
    in_dim = x_ref.shape[1]
    hid = h_ref.shape[1]

    @pl.when(pl.program_id(0) == 0)
    def _fold_weights():
        wcat_ref[:, :hid] = (wz_ref[0, 0, :in_dim, :]
                             + wz_ref[1, 0, :in_dim, :])
        wcat_ref[:, hid:] = 2.0 * (wh_ref[0, 0, :in_dim, :]
                                   + wh_ref[1, 0, :in_dim, :])
        bcat_ref[:, :hid] = bz_ref[...]
        bcat_ref[:, hid:] = 2.0 * bh_ref[...]

    xb = x_ref[...]
    s = jax.nn.sigmoid(
        jnp.dot(xb, wcat_ref[...], preferred_element_type=jnp.float32)
        + bcat_ref[...])
    z = s[:, :hid]
    h_tilde = 2.0 * s[:, hid:] - 1.0
    h = (1.0 - z) * h_tilde
    h_ref[...] = h
    out_ref[...] = (
        jnp.dot(jnp.maximum(h, 0.0), wl_ref[...],
                preferred_element_type=jnp.float32)
        + bl_ref[...])


def kernel(x, edge_index, edge_weight, W_z, b_z, W_r, b_r, W_h, b_h,
           W_lin, b_lin):
    n, in_dim = x.shape
    cat_dim = W_z.shape[2]
    hid = W_z.shape[-1]
    out_dim = W_lin.shape[-1]

    out, h = pl.pallas_call(
        _body,
        grid=(pl.cdiv(n, _ROW_TILE),),
        in_specs=[
            pl.BlockSpec((_ROW_TILE, in_dim), lambda i: (i, 0)),
            pl.BlockSpec((2, 1, cat_dim, hid), lambda i: (0, 0, 0, 0)),
            pl.BlockSpec((1, hid), lambda i: (0, 0)),
            pl.BlockSpec((2, 1, cat_dim, hid), lambda i: (0, 0, 0, 0)),
            pl.BlockSpec((1, hid), lambda i: (0, 0)),
            pl.BlockSpec((hid, out_dim), lambda i: (0, 0)),
            pl.BlockSpec((1, out_dim), lambda i: (0, 0)),
        ],
        out_specs=[
            pl.BlockSpec((_ROW_TILE, out_dim), lambda i: (i, 0)),
            pl.BlockSpec((_ROW_TILE, hid), lambda i: (i, 0)),
        ],
        out_shape=[
            jax.ShapeDtypeStruct((n, out_dim), x.dtype),
            jax.ShapeDtypeStruct((n, hid), x.dtype),
        ],
        scratch_shapes=[
            pltpu.VMEM((in_dim, 2 * hid), jnp.float32),
            pltpu.VMEM((1, 2 * hid), jnp.float32),
        ],
    )(x, W_z, b_z.reshape(1, hid), W_h, b_h.reshape(1, hid),
      W_lin, b_lin.reshape(1, out_dim))
    return (out, h)


# speedup vs baseline: 1.0534x; 1.0033x over previous
import jax
import jax.numpy as jnp
from jax.experimental import pallas as pl
from jax.experimental.pallas import tpu as pltpu

_ROW_TILE = 5008


def _body(x_ref, wz_ref, bz_ref, wh_ref, bh_ref, wl_ref, bl_ref,
          out_ref, h_ref, wcat_ref, bcat_ref):
    in_dim = x_ref.shape[1]
    hid = h_ref.shape[1]

    wcat_ref[:, :hid] = (wz_ref[0, 0, :in_dim, :]
                         + wz_ref[1, 0, :in_dim, :])
    wcat_ref[:, hid:] = 2.0 * (wh_ref[0, 0, :in_dim, :]
                               + wh_ref[1, 0, :in_dim, :])
    bcat_ref[:, :hid] = bz_ref[...]
    bcat_ref[:, hid:] = 2.0 * bh_ref[...]

    xb = x_ref[...]
    s = jax.nn.sigmoid(
        jnp.dot(xb, wcat_ref[...], preferred_element_type=jnp.float32)
        + bcat_ref[...])
    z = s[:, :hid]
    h_tilde = 2.0 * s[:, hid:] - 1.0
    h = (1.0 - z) * h_tilde
    h_ref[...] = h
    out_ref[...] = (
        jnp.dot(jnp.maximum(h, 0.0), wl_ref[...],
                preferred_element_type=jnp.float32)
        + bl_ref[...])


def kernel(x, edge_index, edge_weight, W_z, b_z, W_r, b_r, W_h, b_h,
           W_lin, b_lin):
    n, in_dim = x.shape
    cat_dim = W_z.shape[2]
    hid = W_z.shape[-1]
    out_dim = W_lin.shape[-1]

    out, h = pl.pallas_call(
        _body,
        grid=(pl.cdiv(n, _ROW_TILE),),
        in_specs=[
            pl.BlockSpec((_ROW_TILE, in_dim), lambda i: (i, 0)),
            pl.BlockSpec((2, 1, cat_dim, hid), lambda i: (0, 0, 0, 0)),
            pl.BlockSpec((1, hid), lambda i: (0, 0)),
            pl.BlockSpec((2, 1, cat_dim, hid), lambda i: (0, 0, 0, 0)),
            pl.BlockSpec((1, hid), lambda i: (0, 0)),
            pl.BlockSpec((hid, out_dim), lambda i: (0, 0)),
            pl.BlockSpec((1, out_dim), lambda i: (0, 0)),
        ],
        out_specs=[
            pl.BlockSpec((_ROW_TILE, out_dim), lambda i: (i, 0)),
            pl.BlockSpec((_ROW_TILE, hid), lambda i: (i, 0)),
        ],
        out_shape=[
            jax.ShapeDtypeStruct((n, out_dim), x.dtype),
            jax.ShapeDtypeStruct((n, hid), x.dtype),
        ],
        scratch_shapes=[
            pltpu.VMEM((in_dim, 2 * hid), jnp.float32),
            pltpu.VMEM((1, 2 * hid), jnp.float32),
        ],
    )(x, W_z, b_z.reshape(1, hid), W_h, b_h.reshape(1, hid),
      W_lin, b_lin.reshape(1, out_dim))
    return (out, h)
